# baseline (speedup 1.0000x reference)
import jax
import jax.numpy as jnp
from jax import lax
from jax.experimental import pallas as pl
from jax.experimental.pallas import tpu as pltpu
from jax.experimental.pallas import tpu_sc as plsc

_N, _C = 65536, 256
_NW = 32
_ROWS_PER_W = _N // _NW


def _sc_body(x_hbm, out_hbm, patch_v, sem_big, sem_sm):
    wid = lax.axis_index("s") * 2 + lax.axis_index("c")
    base = wid * _ROWS_PER_W

    src = x_hbm.at[pl.ds(base, _ROWS_PER_W)]
    c1 = pltpu.async_copy(src, out_hbm.at[pl.ds(base, _ROWS_PER_W)], sem_big)
    c2 = pltpu.async_copy(src, out_hbm.at[pl.ds(_N + base, _ROWS_PER_W)],
                          sem_big)
    c1.wait()
    c2.wait()

    @pl.when(wid == 0)
    def _patch():
        pltpu.async_copy(x_hbm.at[0, pl.ds(0, 16)], patch_v, sem_sm).wait()
        lane = lax.iota(jnp.int32, 16)
        patch_v[...] = jnp.where(lane == 0, jnp.float32(100.0),
                                 patch_v[...])
        p1 = pltpu.async_copy(patch_v, out_hbm.at[0, pl.ds(0, 16)], sem_sm)
        p2 = pltpu.async_copy(patch_v, out_hbm.at[_N, pl.ds(0, 16)], sem_sm)
        p1.wait()
        p2.wait()


def kernel(x):
    mesh = plsc.VectorSubcoreMesh(core_axis_name="c", subcore_axis_name="s")
    f = pl.kernel(
        _sc_body,
        out_type=jax.ShapeDtypeStruct((2 * _N, _C), jnp.float32),
        mesh=mesh,
        scratch_types=[
            pltpu.VMEM((16,), jnp.float32),
            pltpu.SemaphoreType.DMA,
            pltpu.SemaphoreType.DMA,
        ],
    )
    return f(x)


# speedup vs baseline: 45.2285x; 45.2285x over previous
import jax
import jax.numpy as jnp
from jax import lax
from jax.experimental import pallas as pl
from jax.experimental.pallas import tpu as pltpu
from jax.experimental.pallas import tpu_sc as plsc

_N, _C = 65536, 256
_NW = 32
_ROWS_PER_W = _N // _NW
_R = 128
_NCH = _ROWS_PER_W // _R


def _sc_body(x_hbm, out_hbm, b0, b1, ld0, ld1, st0, st1):
    wid = lax.axis_index("s") * 2 + lax.axis_index("c")
    base = wid * _ROWS_PER_W
    bufs = (b0, b1)
    lds = (ld0, ld1)
    sts = (st0, st1)

    for b in range(2):
        pltpu.make_async_copy(
            x_hbm.at[pl.ds(base + b * _R, _R)], bufs[b], lds[b]).start()

    def g_body(g, carry):
        for b in range(2):
            k = 2 * g + b
            row = base + k * _R
            pltpu.make_async_copy(
                x_hbm.at[pl.ds(row, _R)], bufs[b], lds[b]).wait()

            @pl.when(jnp.logical_and(wid == 0, k == 0))
            def _patch(b=b):
                v = bufs[b][0, pl.ds(0, 16)]
                lane = lax.iota(jnp.int32, 16)
                bufs[b][0, pl.ds(0, 16)] = jnp.where(
                    lane == 0, jnp.float32(100.0), v)

            pltpu.make_async_copy(
                bufs[b], out_hbm.at[pl.ds(row, _R)], sts[b]).start()
            pltpu.make_async_copy(
                bufs[b], out_hbm.at[pl.ds(_N + row, _R)], sts[b]).start()

            @pl.when(k + 2 < _NCH)
            def _reload(b=b, row=row):
                pltpu.make_async_copy(
                    bufs[b], out_hbm.at[pl.ds(row, _R)], sts[b]).wait()
                pltpu.make_async_copy(
                    bufs[b], out_hbm.at[pl.ds(_N + row, _R)], sts[b]).wait()
                pltpu.make_async_copy(
                    x_hbm.at[pl.ds(row + 2 * _R, _R)], bufs[b],
                    lds[b]).start()
        return carry

    lax.fori_loop(0, _NCH // 2, g_body, 0)

    for b in range(2):
        row = base + (_NCH - 2 + b) * _R
        pltpu.make_async_copy(
            bufs[b], out_hbm.at[pl.ds(row, _R)], sts[b]).wait()
        pltpu.make_async_copy(
            bufs[b], out_hbm.at[pl.ds(_N + row, _R)], sts[b]).wait()


def kernel(x):
    mesh = plsc.VectorSubcoreMesh(core_axis_name="c", subcore_axis_name="s")
    f = pl.kernel(
        _sc_body,
        out_type=jax.ShapeDtypeStruct((2 * _N, _C), jnp.float32),
        mesh=mesh,
        scratch_types=[
            pltpu.VMEM((_R, _C), jnp.float32),
            pltpu.VMEM((_R, _C), jnp.float32),
            pltpu.SemaphoreType.DMA,
            pltpu.SemaphoreType.DMA,
            pltpu.SemaphoreType.DMA,
            pltpu.SemaphoreType.DMA,
        ],
    )
    return f(x)


# speedup vs baseline: 46.5088x; 1.0283x over previous
import jax
import jax.numpy as jnp
from jax import lax
from jax.experimental import pallas as pl
from jax.experimental.pallas import tpu as pltpu
from jax.experimental.pallas import tpu_sc as plsc

_N, _C = 65536, 256
_NW = 32
_ROWS_PER_W = _N // _NW
_R = 248
_CHUNKS = [(i * _R, _R) for i in range(8)] + [(8 * _R, _ROWS_PER_W - 8 * _R)]


def _sc_body(x_hbm, out_hbm, b0, b1, ld0, ld1, st0, st1):
    wid = lax.axis_index("s") * 2 + lax.axis_index("c")
    base = wid * _ROWS_PER_W
    bufs = (b0, b1)
    lds = (ld0, ld1)
    sts = (st0, st1)

    def ld_copy(idx):
        off, ln = _CHUNKS[idx]
        b = idx % 2
        return pltpu.make_async_copy(
            x_hbm.at[pl.ds(base + off, ln)],
            bufs[b].at[pl.ds(0, ln)], lds[b])

    def st_copies(idx):
        off, ln = _CHUNKS[idx]
        b = idx % 2
        return (
            pltpu.make_async_copy(
                bufs[b].at[pl.ds(0, ln)],
                out_hbm.at[pl.ds(base + off, ln)], sts[b]),
            pltpu.make_async_copy(
                bufs[b].at[pl.ds(0, ln)],
                out_hbm.at[pl.ds(_N + base + off, ln)], sts[b]),
        )

    ld_copy(0).start()
    ld_copy(1).start()

    for idx in range(len(_CHUNKS)):
        ld_copy(idx).wait()

        if idx == 0:
            @pl.when(wid == 0)
            def _patch():
                v = b0[0, pl.ds(0, 16)]
                lane = lax.iota(jnp.int32, 16)
                b0[0, pl.ds(0, 16)] = jnp.where(
                    lane == 0, jnp.float32(100.0), v)

        s1, s2 = st_copies(idx)
        s1.start()
        s2.start()

        if idx + 2 < len(_CHUNKS):
            s1.wait()
            s2.wait()
            ld_copy(idx + 2).start()

    for idx in (len(_CHUNKS) - 2, len(_CHUNKS) - 1):
        s1, s2 = st_copies(idx)
        s1.wait()
        s2.wait()


def kernel(x):
    mesh = plsc.VectorSubcoreMesh(core_axis_name="c", subcore_axis_name="s")
    f = pl.kernel(
        _sc_body,
        out_type=jax.ShapeDtypeStruct((2 * _N, _C), jnp.float32),
        mesh=mesh,
        scratch_types=[
            pltpu.VMEM((_R, _C), jnp.float32),
            pltpu.VMEM((_R, _C), jnp.float32),
            pltpu.SemaphoreType.DMA,
            pltpu.SemaphoreType.DMA,
            pltpu.SemaphoreType.DMA,
            pltpu.SemaphoreType.DMA,
        ],
    )
    return f(x)


# speedup vs baseline: 49.4492x; 1.0632x over previous
import jax
import jax.numpy as jnp
from jax import lax
from jax.experimental import pallas as pl
from jax.experimental.pallas import tpu as pltpu
from jax.experimental.pallas import tpu_sc as plsc

_N, _C = 65536, 256

_NW = 32
_TEC_ROWS = 36608
_TEC_PER_W = _TEC_ROWS // _NW
_TR = 248

_NSCS = 2
_SCS_ROWS = _N - _TEC_ROWS
_SCS_PER_W = _SCS_ROWS // _NSCS
_SR = 2048


def _chunk_list(total, step):
    return [(o, min(step, total - o)) for o in range(0, total, step)]

_TCH = _chunk_list(_TEC_PER_W, _TR)
_SCH = _chunk_list(_SCS_PER_W, _SR)


def _ring(x_hbm, out_hbm, base, chunks, bufs, lds, sts, patch_wid=None):
    def ld_copy(idx):
        off, ln = chunks[idx]
        b = idx % 2
        return pltpu.make_async_copy(
            x_hbm.at[pl.ds(base + off, ln)],
            bufs[b].at[pl.ds(0, ln)], lds[b])

    def st_copies(idx):
        off, ln = chunks[idx]
        b = idx % 2
        return (
            pltpu.make_async_copy(
                bufs[b].at[pl.ds(0, ln)],
                out_hbm.at[pl.ds(base + off, ln)], sts[b]),
            pltpu.make_async_copy(
                bufs[b].at[pl.ds(0, ln)],
                out_hbm.at[pl.ds(_N + base + off, ln)], sts[b]),
        )

    ld_copy(0).start()
    if len(chunks) > 1:
        ld_copy(1).start()

    for idx in range(len(chunks)):
        ld_copy(idx).wait()

        if idx == 0 and patch_wid is not None:
            @pl.when(patch_wid == 0)
            def _patch():
                v = bufs[0][0, pl.ds(0, 16)]
                lane = lax.iota(jnp.int32, 16)
                bufs[0][0, pl.ds(0, 16)] = jnp.where(
                    lane == 0, jnp.float32(100.0), v)

        s1, s2 = st_copies(idx)
        s1.start()
        s2.start()

        if idx + 2 < len(chunks):
            s1.wait()
            s2.wait()
            ld_copy(idx + 2).start()

    for idx in range(max(0, len(chunks) - 2), len(chunks)):
        s1, s2 = st_copies(idx)
        s1.wait()
        s2.wait()


def _tec_body(x_hbm, out_hbm):
    wid = lax.axis_index("s") * 2 + lax.axis_index("c")
    base = wid * _TEC_PER_W

    def scoped(b0, b1, ld0, ld1, st0, st1):
        _ring(x_hbm, out_hbm, base, _TCH, (b0, b1), (ld0, ld1), (st0, st1),
              patch_wid=wid)

    pl.run_scoped(
        scoped,
        pltpu.VMEM((_TR, _C), jnp.float32),
        pltpu.VMEM((_TR, _C), jnp.float32),
        pltpu.SemaphoreType.DMA,
        pltpu.SemaphoreType.DMA,
        pltpu.SemaphoreType.DMA,
        pltpu.SemaphoreType.DMA,
    )


def _scs_body(x_hbm, out_hbm):
    cid = lax.axis_index("c")
    base = _TEC_ROWS + cid * _SCS_PER_W

    def scoped(b0, b1, ld0, ld1, st0, st1):
        _ring(x_hbm, out_hbm, base, _SCH, (b0, b1), (ld0, ld1), (st0, st1))

    pl.run_scoped(
        scoped,
        pltpu.VMEM_SHARED((_SR, _C), jnp.float32),
        pltpu.VMEM_SHARED((_SR, _C), jnp.float32),
        pltpu.SemaphoreType.DMA,
        pltpu.SemaphoreType.DMA,
        pltpu.SemaphoreType.DMA,
        pltpu.SemaphoreType.DMA,
    )


def kernel(x):
    v_mesh = plsc.VectorSubcoreMesh(core_axis_name="c",
                                    subcore_axis_name="s")
    s_mesh = plsc.ScalarSubcoreMesh(axis_name="c", num_cores=_NSCS)
    f = pl.kernel(
        body=[_tec_body, _scs_body],
        mesh=[v_mesh, s_mesh],
        out_type=jax.ShapeDtypeStruct((2 * _N, _C), jnp.float32),
    )
    return f(x)
